# baseline (speedup 1.0000x reference)
import functools

import jax
import jax.numpy as jnp
from jax import lax
from jax.experimental import pallas as pl
from jax.experimental.pallas import tpu as pltpu
from jax.experimental.pallas import tpu_sc as plsc

N = 10000
E = 320000
IN_DIM = 128
HID = 64

NC = 2
NS = 16
NW = NC * NS
L = 16

C = 128
K = 80
EPW = K * C
E_PAD = NW * EPW
N_PAD = 10240
RPS = N_PAD // NS

f32 = jnp.float32


def _make_seg_sum(with_counts: bool):
    out_type = [jax.ShapeDtypeStruct((NC, N_PAD, HID), f32)]
    scratch = [
        pltpu.VMEM((K + 1, C), jnp.int32),
        pltpu.VMEM((K, C), jnp.int32),
        pltpu.VMEM((C, HID), f32),
        pltpu.VMEM((C, HID), f32),
        pltpu.VMEM((C, HID), f32),
        pltpu.VMEM_SHARED((N_PAD, HID), f32),
        pltpu.SemaphoreType.DMA,
        pltpu.SemaphoreType.DMA,
    ]
    if with_counts:
        out_type.append(jax.ShapeDtypeStruct((NC, N_PAD), f32))
        scratch += [
            pltpu.VMEM((C,), f32),
            pltpu.VMEM((RPS,), f32),
            pltpu.VMEM_SHARED((N_PAD,), f32),
        ]

    mesh = plsc.VectorSubcoreMesh(core_axis_name="c", subcore_axis_name="s")

    def body(table, srcs, dsts, *refs):
        if with_counts:
            (out_sums, out_cnts, src_v, dst_v, rows0_v, rows1_v, zrow_v,
             acc_sh, sem0, sem1, ones_v, zcnt_v, cnt_sh) = refs
        else:
            (out_sums, src_v, dst_v, rows0_v, rows1_v, zrow_v,
             acc_sh, sem0, sem1) = refs
        cid = lax.axis_index("c")
        sid = lax.axis_index("s")
        wid = sid * NC + cid

        zv = jnp.zeros((L,), f32)

        def zrow_body(i, _):
            for j in range(HID // L):
                zrow_v[i, pl.ds(j * L, L)] = zv
            return _

        lax.fori_loop(0, C, zrow_body, None)
        if with_counts:
            ov = jnp.ones((L,), f32)
            for j in range(C // L):
                ones_v[pl.ds(j * L, L)] = ov
            for j in range(RPS // L):
                zcnt_v[pl.ds(j * L, L)] = zv

        for b in range(RPS // C):
            pltpu.sync_copy(zrow_v, acc_sh.at[pl.ds(sid * RPS + b * C, C)])
        if with_counts:
            pltpu.sync_copy(zcnt_v, cnt_sh.at[pl.ds(sid * RPS, RPS)])
        plsc.subcore_barrier()

        pltpu.sync_copy(srcs.at[wid], src_v.at[pl.ds(0, K)])
        pltpu.sync_copy(srcs.at[wid, pl.ds(0, 1)], src_v.at[pl.ds(K, 1)])
        pltpu.sync_copy(dsts.at[wid], dst_v)

        pltpu.async_copy(table.at[src_v.at[0]], rows0_v, sem0)

        def chunk_pair(i, _):
            k0 = 2 * i
            pltpu.make_async_copy(table.at[src_v.at[k0]], rows0_v,
                                  sem0).wait()
            pltpu.async_copy(table.at[src_v.at[k0 + 1]], rows1_v, sem1)
            pltpu.sync_copy(rows0_v, acc_sh.at[dst_v.at[k0]], add=True)
            if with_counts:
                pltpu.sync_copy(ones_v, cnt_sh.at[dst_v.at[k0]], add=True)
            pltpu.make_async_copy(table.at[src_v.at[k0 + 1]], rows1_v,
                                  sem1).wait()
            pltpu.async_copy(table.at[src_v.at[k0 + 2]], rows0_v, sem0)
            pltpu.sync_copy(rows1_v, acc_sh.at[dst_v.at[k0 + 1]], add=True)
            if with_counts:
                pltpu.sync_copy(ones_v, cnt_sh.at[dst_v.at[k0 + 1]],
                                add=True)
            return _

        lax.fori_loop(0, K // 2, chunk_pair, None)
        pltpu.make_async_copy(table.at[src_v.at[K]], rows0_v, sem0).wait()
        plsc.subcore_barrier()

        rows = pl.ds(sid * RPS, RPS)
        pltpu.sync_copy(acc_sh.at[rows], out_sums.at[cid, rows])
        if with_counts:
            pltpu.sync_copy(cnt_sh.at[rows], out_cnts.at[cid, rows])

    return pl.kernel(body, out_type=tuple(out_type), mesh=mesh,
                     scratch_types=scratch,
                     compiler_params=pltpu.CompilerParams(
                         use_tc_tiling_on_sc=False))


_seg_sum_cnt = _make_seg_sum(with_counts=True)
_seg_sum = _make_seg_sum(with_counts=False)


_DN = (((1,), (1,)), ((), ()))


def _lin1_body(x_ref, wl_ref, wr_ref, xl_ref, xr_ref):
    x = x_ref[...]
    xl_ref[...] = lax.dot_general(x, wl_ref[...], _DN,
                                  preferred_element_type=f32)
    xr_ref[...] = lax.dot_general(x, wr_ref[...], _DN,
                                  preferred_element_type=f32)


def _mid_body(s_ref, c_ref, xr_ref, b1_ref, w2l_ref, w2r_ref,
              hl_ref, hr_ref):
    s = s_ref[0, ...] + s_ref[1, ...]
    c = jnp.maximum(c_ref[0, ...] + c_ref[1, ...], 1.0)
    h = jnp.maximum(s / c + b1_ref[...] + xr_ref[...], 0.0)
    hl_ref[...] = lax.dot_general(h, w2l_ref[...], _DN,
                                  preferred_element_type=f32)
    hr_ref[...] = lax.dot_general(h, w2r_ref[...], _DN,
                                  preferred_element_type=f32)


def _head_body(s_ref, c_ref, hr_ref, b2_ref, wh_ref, bh_ref, out_ref):
    s = s_ref[0, ...] + s_ref[1, ...]
    c = jnp.maximum(c_ref[0, ...] + c_ref[1, ...], 1.0)
    h2 = jnp.maximum(s / c + b2_ref[...] + hr_ref[...], 0.0)
    out_ref[...] = lax.dot_general(h2, wh_ref[...], _DN,
                                   preferred_element_type=f32) + bh_ref[0, 0]


def kernel(x, edge_index, W1l, b1l, W1r, W2l, b2l, W2r, Wh, bh):
    src = edge_index[0].astype(jnp.int32)
    dst = edge_index[1].astype(jnp.int32)
    pad = E_PAD - E
    src_p = jnp.concatenate([src, jnp.full((pad,), N, jnp.int32)])
    dst_p = jnp.concatenate([dst, jnp.full((pad,), N, jnp.int32)])
    srcs = src_p.reshape(NW, K, C)
    dsts = dst_p.reshape(NW, K, C)

    x_pad = jnp.concatenate([x, jnp.zeros((N_PAD - N, IN_DIM), f32)])

    xl, xr = pl.pallas_call(
        _lin1_body,
        out_shape=(jax.ShapeDtypeStruct((N_PAD, HID), f32),
                   jax.ShapeDtypeStruct((N_PAD, HID), f32)),
    )(x_pad, W1l, W1r)

    sums1, cnts = _seg_sum_cnt(xl, srcs, dsts)
    cnts3 = cnts.reshape(NC, N_PAD, 1)

    hl, hr = pl.pallas_call(
        _mid_body,
        out_shape=(jax.ShapeDtypeStruct((N_PAD, HID), f32),
                   jax.ShapeDtypeStruct((N_PAD, HID), f32)),
    )(sums1, cnts3, xr, b1l.reshape(1, HID), W2l, W2r)

    (sums2,) = _seg_sum(hl, srcs, dsts)

    Wh_p = jnp.concatenate([Wh, jnp.zeros((7, HID), f32)])
    logits = pl.pallas_call(
        _head_body,
        out_shape=jax.ShapeDtypeStruct((N_PAD, 8), f32),
    )(sums2, cnts3, hr, b2l.reshape(1, HID), Wh_p, bh.reshape(1, 1))

    return logits[:N, 0]


# speedup vs baseline: 1.0023x; 1.0023x over previous
import functools

import jax
import jax.numpy as jnp
from jax import lax
from jax.experimental import pallas as pl
from jax.experimental.pallas import tpu as pltpu
from jax.experimental.pallas import tpu_sc as plsc

N = 10000
E = 320000
IN_DIM = 128
HID = 64

NC = 2
NS = 16
NW = NC * NS
L = 16

C = 128
K = 80
G = 2
SEG = 2 * G
EPW = K * C
E_PAD = NW * EPW
N_PAD = 10240
RPS = N_PAD // NS

f32 = jnp.float32


def _make_seg_sum(with_counts: bool):
    out_type = [jax.ShapeDtypeStruct((NC, N_PAD, HID), f32)]
    scratch = [
        pltpu.VMEM((K, C), jnp.int32),
        pltpu.VMEM((K, C), jnp.int32),
    ] + [pltpu.VMEM((C, HID), f32) for _ in range(SEG)] + [
        pltpu.VMEM((C, HID), f32),
        pltpu.VMEM_SHARED((N_PAD, HID), f32),
        pltpu.SemaphoreType.DMA,
        pltpu.SemaphoreType.DMA,
    ]
    if with_counts:
        out_type.append(jax.ShapeDtypeStruct((NC, N_PAD), f32))
        scratch += [
            pltpu.VMEM((C,), f32),
            pltpu.VMEM((RPS,), f32),
            pltpu.VMEM_SHARED((N_PAD,), f32),
        ]

    mesh = plsc.VectorSubcoreMesh(core_axis_name="c", subcore_axis_name="s")

    def body(table, srcs, dsts, *refs):
        if with_counts:
            (out_sums, out_cnts, src_v, dst_v, *rows,
             zrow_v, acc_sh, semg, sems, ones_v, zcnt_v, cnt_sh) = refs
        else:
            (out_sums, src_v, dst_v, *rows,
             zrow_v, acc_sh, semg, sems) = refs
        assert len(rows) == SEG
        cid = lax.axis_index("c")
        sid = lax.axis_index("s")
        wid = sid * NC + cid

        zv = jnp.zeros((L,), f32)

        def zrow_body(i, _):
            for j in range(HID // L):
                zrow_v[i, pl.ds(j * L, L)] = zv
            return _

        lax.fori_loop(0, C, zrow_body, None)
        if with_counts:
            ov = jnp.ones((L,), f32)
            for j in range(C // L):
                ones_v[pl.ds(j * L, L)] = ov
            for j in range(RPS // L):
                zcnt_v[pl.ds(j * L, L)] = zv

        for b in range(RPS // C):
            pltpu.sync_copy(zrow_v, acc_sh.at[pl.ds(sid * RPS + b * C, C)])
        if with_counts:
            pltpu.sync_copy(zcnt_v, cnt_sh.at[pl.ds(sid * RPS, RPS)])
        plsc.subcore_barrier()

        pltpu.sync_copy(srcs.at[wid], src_v)
        pltpu.sync_copy(dsts.at[wid], dst_v)

        def scatter_group(base, lo):
            descs = []
            for j in range(G):
                descs.append(pltpu.async_copy(
                    rows[lo + j], acc_sh.at[dst_v.at[base + j]], sems,
                    add=True))
                if with_counts:
                    descs.append(pltpu.async_copy(
                        ones_v, cnt_sh.at[dst_v.at[base + j]], sems,
                        add=True))
            return descs

        def seg_body(i, _):
            base = i * SEG
            ga = [pltpu.async_copy(table.at[src_v.at[base + j]], rows[j],
                                   semg) for j in range(G)]
            for d in ga:
                d.wait()
            gb = [pltpu.async_copy(table.at[src_v.at[base + G + j]],
                                   rows[G + j], semg) for j in range(G)]
            sa = scatter_group(base, 0)
            for d in gb:
                d.wait()
            sb = scatter_group(base + G, G)
            for d in sa + sb:
                d.wait()
            return _

        lax.fori_loop(0, K // SEG, seg_body, None)
        plsc.subcore_barrier()

        rows = pl.ds(sid * RPS, RPS)
        pltpu.sync_copy(acc_sh.at[rows], out_sums.at[cid, rows])
        if with_counts:
            pltpu.sync_copy(cnt_sh.at[rows], out_cnts.at[cid, rows])

    return pl.kernel(body, out_type=tuple(out_type), mesh=mesh,
                     scratch_types=scratch,
                     compiler_params=pltpu.CompilerParams(
                         use_tc_tiling_on_sc=False))


_seg_sum_cnt = _make_seg_sum(with_counts=True)
_seg_sum = _make_seg_sum(with_counts=False)


_DN = (((1,), (1,)), ((), ()))


def _lin1_body(x_ref, wl_ref, wr_ref, xl_ref, xr_ref):
    x = x_ref[...]
    xl_ref[...] = lax.dot_general(x, wl_ref[...], _DN,
                                  preferred_element_type=f32)
    xr_ref[...] = lax.dot_general(x, wr_ref[...], _DN,
                                  preferred_element_type=f32)


def _mid_body(s_ref, c_ref, xr_ref, b1_ref, w2l_ref, w2r_ref,
              hl_ref, hr_ref):
    s = s_ref[0, ...] + s_ref[1, ...]
    c = jnp.maximum(c_ref[0, ...] + c_ref[1, ...], 1.0)
    h = jnp.maximum(s / c + b1_ref[...] + xr_ref[...], 0.0)
    hl_ref[...] = lax.dot_general(h, w2l_ref[...], _DN,
                                  preferred_element_type=f32)
    hr_ref[...] = lax.dot_general(h, w2r_ref[...], _DN,
                                  preferred_element_type=f32)


def _head_body(s_ref, c_ref, hr_ref, b2_ref, wh_ref, bh_ref, out_ref):
    s = s_ref[0, ...] + s_ref[1, ...]
    c = jnp.maximum(c_ref[0, ...] + c_ref[1, ...], 1.0)
    h2 = jnp.maximum(s / c + b2_ref[...] + hr_ref[...], 0.0)
    out_ref[...] = lax.dot_general(h2, wh_ref[...], _DN,
                                   preferred_element_type=f32) + bh_ref[0, 0]


def kernel(x, edge_index, W1l, b1l, W1r, W2l, b2l, W2r, Wh, bh):
    src = edge_index[0].astype(jnp.int32)
    dst = edge_index[1].astype(jnp.int32)
    pad = E_PAD - E
    src_p = jnp.concatenate([src, jnp.full((pad,), N, jnp.int32)])
    dst_p = jnp.concatenate([dst, jnp.full((pad,), N, jnp.int32)])
    srcs = src_p.reshape(NW, K, C)
    dsts = dst_p.reshape(NW, K, C)

    x_pad = jnp.concatenate([x, jnp.zeros((N_PAD - N, IN_DIM), f32)])

    xl, xr = pl.pallas_call(
        _lin1_body,
        out_shape=(jax.ShapeDtypeStruct((N_PAD, HID), f32),
                   jax.ShapeDtypeStruct((N_PAD, HID), f32)),
    )(x_pad, W1l, W1r)

    sums1, cnts = _seg_sum_cnt(xl, srcs, dsts)
    cnts3 = cnts.reshape(NC, N_PAD, 1)

    hl, hr = pl.pallas_call(
        _mid_body,
        out_shape=(jax.ShapeDtypeStruct((N_PAD, HID), f32),
                   jax.ShapeDtypeStruct((N_PAD, HID), f32)),
    )(sums1, cnts3, xr, b1l.reshape(1, HID), W2l, W2r)

    (sums2,) = _seg_sum(hl, srcs, dsts)

    Wh_p = jnp.concatenate([Wh, jnp.zeros((7, HID), f32)])
    logits = pl.pallas_call(
        _head_body,
        out_shape=jax.ShapeDtypeStruct((N_PAD, 8), f32),
    )(sums2, cnts3, hr, b2l.reshape(1, HID), Wh_p, bh.reshape(1, 1))

    return logits[:N, 0]


# speedup vs baseline: 1.0250x; 1.0227x over previous
import functools

import jax
import jax.numpy as jnp
from jax import lax
from jax.experimental import pallas as pl
from jax.experimental.pallas import tpu as pltpu
from jax.experimental.pallas import tpu_sc as plsc

N = 10000
E = 320000
IN_DIM = 128
HID = 64

NC = 2
NS = 16
NW = NC * NS
L = 16

C = 128
K = 80
G = 2
SEG = 2 * G
EPW = K * C
E_PAD = NW * EPW
N_PAD = 10240
RPS = N_PAD // NS

f32 = jnp.float32


def _make_seg_sum(with_counts: bool):
    out_type = [jax.ShapeDtypeStruct((NC, N_PAD, HID), f32)]
    scratch = [
        pltpu.VMEM((K, C), jnp.int32),
        pltpu.VMEM((K, C), jnp.int32),
    ] + [pltpu.VMEM((C, HID), f32) for _ in range(SEG)] + [
        pltpu.VMEM((C, HID), f32),
        pltpu.VMEM_SHARED((N_PAD, HID), f32),
        pltpu.SemaphoreType.DMA,
        pltpu.SemaphoreType.DMA,
    ]
    if with_counts:
        out_type.append(jax.ShapeDtypeStruct((NC, N_PAD), f32))
        scratch += [
            pltpu.VMEM((C,), f32),
            pltpu.VMEM((RPS,), f32),
            pltpu.VMEM_SHARED((N_PAD,), f32),
        ]

    mesh = plsc.VectorSubcoreMesh(core_axis_name="c", subcore_axis_name="s")

    def body(table, srcs, dsts, *refs):
        if with_counts:
            (out_sums, out_cnts, src_v, dst_v, *rows,
             zrow_v, acc_sh, semg, sems, ones_v, zcnt_v, cnt_sh) = refs
        else:
            (out_sums, src_v, dst_v, *rows,
             zrow_v, acc_sh, semg, sems) = refs
        assert len(rows) == SEG
        cid = lax.axis_index("c")
        sid = lax.axis_index("s")
        wid = sid * NC + cid

        zv = jnp.zeros((L,), f32)

        def zrow_body(i, _):
            for j in range(HID // L):
                zrow_v[i, pl.ds(j * L, L)] = zv
            return _

        lax.fori_loop(0, C, zrow_body, None)
        if with_counts:
            ov = jnp.ones((L,), f32)
            for j in range(C // L):
                ones_v[pl.ds(j * L, L)] = ov
            for j in range(RPS // L):
                zcnt_v[pl.ds(j * L, L)] = zv

        for b in range(RPS // C):
            pltpu.sync_copy(zrow_v, acc_sh.at[pl.ds(sid * RPS + b * C, C)])
        if with_counts:
            pltpu.sync_copy(zcnt_v, cnt_sh.at[pl.ds(sid * RPS, RPS)])
        plsc.subcore_barrier()

        pltpu.sync_copy(srcs.at[wid], src_v)
        pltpu.sync_copy(dsts.at[wid], dst_v)

        def seg_body(i, _):
            base = i * SEG
            gd = [pltpu.async_copy(table.at[src_v.at[base + j]], rows[j],
                                   semg) for j in range(SEG)]
            for j in range(SEG):
                gd[j].wait()
                pltpu.sync_copy(rows[j], acc_sh.at[dst_v.at[base + j]],
                                add=True)
                if with_counts:
                    pltpu.sync_copy(ones_v, cnt_sh.at[dst_v.at[base + j]],
                                    add=True)
            return _

        lax.fori_loop(0, K // SEG, seg_body, None)
        plsc.subcore_barrier()

        rows = pl.ds(sid * RPS, RPS)
        pltpu.sync_copy(acc_sh.at[rows], out_sums.at[cid, rows])
        if with_counts:
            pltpu.sync_copy(cnt_sh.at[rows], out_cnts.at[cid, rows])

    return pl.kernel(body, out_type=tuple(out_type), mesh=mesh,
                     scratch_types=scratch,
                     compiler_params=pltpu.CompilerParams(
                         use_tc_tiling_on_sc=False))


_seg_sum_cnt = _make_seg_sum(with_counts=True)
_seg_sum = _make_seg_sum(with_counts=False)


_DN = (((1,), (1,)), ((), ()))


def _lin1_body(x_ref, wl_ref, wr_ref, xl_ref, xr_ref):
    x = x_ref[...]
    xl_ref[...] = lax.dot_general(x, wl_ref[...], _DN,
                                  preferred_element_type=f32)
    xr_ref[...] = lax.dot_general(x, wr_ref[...], _DN,
                                  preferred_element_type=f32)


def _mid_body(s_ref, c_ref, xr_ref, b1_ref, w2l_ref, w2r_ref,
              hl_ref, hr_ref):
    s = s_ref[0, ...] + s_ref[1, ...]
    c = jnp.maximum(c_ref[0, ...] + c_ref[1, ...], 1.0)
    h = jnp.maximum(s / c + b1_ref[...] + xr_ref[...], 0.0)
    hl_ref[...] = lax.dot_general(h, w2l_ref[...], _DN,
                                  preferred_element_type=f32)
    hr_ref[...] = lax.dot_general(h, w2r_ref[...], _DN,
                                  preferred_element_type=f32)


def _head_body(s_ref, c_ref, hr_ref, b2_ref, wh_ref, bh_ref, out_ref):
    s = s_ref[0, ...] + s_ref[1, ...]
    c = jnp.maximum(c_ref[0, ...] + c_ref[1, ...], 1.0)
    h2 = jnp.maximum(s / c + b2_ref[...] + hr_ref[...], 0.0)
    out_ref[...] = lax.dot_general(h2, wh_ref[...], _DN,
                                   preferred_element_type=f32) + bh_ref[0, 0]


def kernel(x, edge_index, W1l, b1l, W1r, W2l, b2l, W2r, Wh, bh):
    src = edge_index[0].astype(jnp.int32)
    dst = edge_index[1].astype(jnp.int32)
    pad = E_PAD - E
    src_p = jnp.concatenate([src, jnp.full((pad,), N, jnp.int32)])
    dst_p = jnp.concatenate([dst, jnp.full((pad,), N, jnp.int32)])
    srcs = src_p.reshape(NW, K, C)
    dsts = dst_p.reshape(NW, K, C)

    x_pad = jnp.concatenate([x, jnp.zeros((N_PAD - N, IN_DIM), f32)])

    xl, xr = pl.pallas_call(
        _lin1_body,
        out_shape=(jax.ShapeDtypeStruct((N_PAD, HID), f32),
                   jax.ShapeDtypeStruct((N_PAD, HID), f32)),
    )(x_pad, W1l, W1r)

    sums1, cnts = _seg_sum_cnt(xl, srcs, dsts)
    cnts3 = cnts.reshape(NC, N_PAD, 1)

    hl, hr = pl.pallas_call(
        _mid_body,
        out_shape=(jax.ShapeDtypeStruct((N_PAD, HID), f32),
                   jax.ShapeDtypeStruct((N_PAD, HID), f32)),
    )(sums1, cnts3, xr, b1l.reshape(1, HID), W2l, W2r)

    (sums2,) = _seg_sum(hl, srcs, dsts)

    Wh_p = jnp.concatenate([Wh, jnp.zeros((7, HID), f32)])
    logits = pl.pallas_call(
        _head_body,
        out_shape=jax.ShapeDtypeStruct((N_PAD, 8), f32),
    )(sums2, cnts3, hr, b2l.reshape(1, HID), Wh_p, bh.reshape(1, 1))

    return logits[:N, 0]


# speedup vs baseline: 1.0404x; 1.0150x over previous
import functools

import jax
import jax.numpy as jnp
from jax import lax
from jax.experimental import pallas as pl
from jax.experimental.pallas import tpu as pltpu
from jax.experimental.pallas import tpu_sc as plsc

N = 10000
E = 320000
IN_DIM = 128
HID = 64

NC = 2
NS = 16
NW = NC * NS
L = 16

C = 128
K = 80
G = 2
SEG = 2 * G
EPW = K * C
E_PAD = NW * EPW
N_PAD = 10240
RPS = N_PAD // NS

f32 = jnp.float32


def _make_seg_sum(with_counts: bool):
    out_type = [jax.ShapeDtypeStruct((NC, N_PAD, HID), f32)]
    scratch = [
        pltpu.VMEM((K, C), jnp.int32),
        pltpu.VMEM((K, C), jnp.int32),
        pltpu.VMEM((C, HID), f32),
        pltpu.VMEM((C, HID), f32),
        pltpu.VMEM_SHARED((N_PAD, HID), f32),
        pltpu.SemaphoreType.DMA,
    ]
    if with_counts:
        out_type.append(jax.ShapeDtypeStruct((NC, N_PAD), f32))
        scratch += [
            pltpu.VMEM((C,), f32),
            pltpu.VMEM((RPS,), f32),
            pltpu.VMEM_SHARED((N_PAD,), f32),
        ]

    mesh = plsc.VectorSubcoreMesh(core_axis_name="c", subcore_axis_name="s")

    def body(table, srcs, dsts, *refs):
        if with_counts:
            (out_sums, out_cnts, src_v, dst_v, rows_v,
             zrow_v, acc_sh, semg, ones_v, zcnt_v, cnt_sh) = refs
        else:
            (out_sums, src_v, dst_v, rows_v,
             zrow_v, acc_sh, semg) = refs
        cid = lax.axis_index("c")
        sid = lax.axis_index("s")
        wid = sid * NC + cid

        zv = jnp.zeros((L,), f32)

        def zrow_body(i, _):
            for j in range(HID // L):
                zrow_v[i, pl.ds(j * L, L)] = zv
            return _

        lax.fori_loop(0, C, zrow_body, None)
        if with_counts:
            ov = jnp.ones((L,), f32)
            for j in range(C // L):
                ones_v[pl.ds(j * L, L)] = ov
            for j in range(RPS // L):
                zcnt_v[pl.ds(j * L, L)] = zv

        for b in range(RPS // C):
            pltpu.sync_copy(zrow_v, acc_sh.at[pl.ds(sid * RPS + b * C, C)])
        if with_counts:
            pltpu.sync_copy(zcnt_v, cnt_sh.at[pl.ds(sid * RPS, RPS)])
        plsc.subcore_barrier()

        pltpu.sync_copy(srcs.at[wid], src_v)
        pltpu.sync_copy(dsts.at[wid], dst_v)

        def chunk(k, _):
            pltpu.async_copy(table.at[src_v.at[k]], rows_v, semg).wait()
            pltpu.sync_copy(rows_v, acc_sh.at[dst_v.at[k]], add=True)
            if with_counts:
                pltpu.sync_copy(ones_v, cnt_sh.at[dst_v.at[k]], add=True)
            return _

        lax.fori_loop(0, K, chunk, None)
        plsc.subcore_barrier()

        rows = pl.ds(sid * RPS, RPS)
        pltpu.sync_copy(acc_sh.at[rows], out_sums.at[cid, rows])
        if with_counts:
            pltpu.sync_copy(cnt_sh.at[rows], out_cnts.at[cid, rows])

    return pl.kernel(body, out_type=tuple(out_type), mesh=mesh,
                     scratch_types=scratch,
                     compiler_params=pltpu.CompilerParams(
                         use_tc_tiling_on_sc=False))


_seg_sum_cnt = _make_seg_sum(with_counts=True)
_seg_sum = _make_seg_sum(with_counts=False)


_DN = (((1,), (1,)), ((), ()))


def _lin1_body(x_ref, wl_ref, wr_ref, xl_ref, xr_ref):
    x = x_ref[...]
    xl_ref[...] = lax.dot_general(x, wl_ref[...], _DN,
                                  preferred_element_type=f32)
    xr_ref[...] = lax.dot_general(x, wr_ref[...], _DN,
                                  preferred_element_type=f32)


def _mid_body(s_ref, c_ref, xr_ref, b1_ref, w2l_ref, w2r_ref,
              hl_ref, hr_ref):
    s = s_ref[0, pl.ds(0, N), :] + s_ref[1, pl.ds(0, N), :]
    c = jnp.maximum(c_ref[0, pl.ds(0, N)] + c_ref[1, pl.ds(0, N)], 1.0)
    h = jnp.maximum(s / c[:, None] + b1_ref[...] + xr_ref[...], 0.0)
    hl_ref[...] = lax.dot_general(h, w2l_ref[...], _DN,
                                  preferred_element_type=f32)
    hr_ref[...] = lax.dot_general(h, w2r_ref[...], _DN,
                                  preferred_element_type=f32)


def _head_body(s_ref, c_ref, hr_ref, b2_ref, wh_ref, bh_ref, out_ref):
    s = s_ref[0, pl.ds(0, N), :] + s_ref[1, pl.ds(0, N), :]
    c = jnp.maximum(c_ref[0, pl.ds(0, N)] + c_ref[1, pl.ds(0, N)], 1.0)
    h2 = jnp.maximum(s / c[:, None] + b2_ref[...] + hr_ref[...], 0.0)
    out_ref[...] = lax.dot_general(h2, wh_ref[...], _DN,
                                   preferred_element_type=f32) + bh_ref[0, 0]


def kernel(x, edge_index, W1l, b1l, W1r, W2l, b2l, W2r, Wh, bh):
    src = edge_index[0].astype(jnp.int32)
    dst = edge_index[1].astype(jnp.int32)
    pad = E_PAD - E
    src_p = jnp.concatenate([src, jnp.zeros((pad,), jnp.int32)])
    dst_p = jnp.concatenate([dst, jnp.full((pad,), N, jnp.int32)])
    srcs = src_p.reshape(NW, K, C)
    dsts = dst_p.reshape(NW, K, C)

    xl, xr = pl.pallas_call(
        _lin1_body,
        out_shape=(jax.ShapeDtypeStruct((N, HID), f32),
                   jax.ShapeDtypeStruct((N, HID), f32)),
    )(x, W1l, W1r)

    sums1, cnts = _seg_sum_cnt(xl, srcs, dsts)

    hl, hr = pl.pallas_call(
        _mid_body,
        out_shape=(jax.ShapeDtypeStruct((N, HID), f32),
                   jax.ShapeDtypeStruct((N, HID), f32)),
    )(sums1, cnts, xr, b1l.reshape(1, HID), W2l, W2r)

    (sums2,) = _seg_sum(hl, srcs, dsts)

    Wh_p = jnp.concatenate([Wh, jnp.zeros((7, HID), f32)])
    logits = pl.pallas_call(
        _head_body,
        out_shape=jax.ShapeDtypeStruct((N, 8), f32),
    )(sums2, cnts, hr, b2l.reshape(1, HID), Wh_p, bh.reshape(1, 1))

    return logits[:, 0]


# speedup vs baseline: 1.3494x; 1.2971x over previous
import functools

import jax
import jax.numpy as jnp
from jax import lax
from jax.experimental import pallas as pl
from jax.experimental.pallas import tpu as pltpu
from jax.experimental.pallas import tpu_sc as plsc

N = 10000
E = 320000
IN_DIM = 128
HID = 64

NC = 2
NS = 16
NW = NC * NS
L = 16

C = 128
K = 79
G = 2
SEG = 2 * G
EPW = K * C
E_PAD = NW * EPW
N_PAD = 10240
RPS = N_PAD // NS

f32 = jnp.float32


def _make_seg_sum(with_counts: bool):
    out_type = [jax.ShapeDtypeStruct((NC, N_PAD, HID), f32)]
    scratch = [
        pltpu.VMEM((K, C), jnp.int32),
        pltpu.VMEM((K, C), jnp.int32),
        pltpu.VMEM((C, HID), f32),
        pltpu.VMEM((C, HID), f32),
        pltpu.VMEM_SHARED((N_PAD, HID), f32),
        pltpu.SemaphoreType.DMA,
    ]
    if with_counts:
        out_type.append(jax.ShapeDtypeStruct((NC, N_PAD), f32))
        scratch += [
            pltpu.VMEM((C,), f32),
            pltpu.VMEM((RPS,), f32),
            pltpu.VMEM_SHARED((N_PAD,), f32),
        ]

    mesh = plsc.VectorSubcoreMesh(core_axis_name="c", subcore_axis_name="s")

    def body(table, srcs, dsts, *refs):
        if with_counts:
            (out_sums, out_cnts, src_v, dst_v, rows_v,
             zrow_v, acc_sh, semg, ones_v, zcnt_v, cnt_sh) = refs
        else:
            (out_sums, src_v, dst_v, rows_v,
             zrow_v, acc_sh, semg) = refs
        cid = lax.axis_index("c")
        sid = lax.axis_index("s")
        wid = sid * NC + cid

        zv = jnp.zeros((L,), f32)

        def zrow_body(i, _):
            for j in range(HID // L):
                zrow_v[i, pl.ds(j * L, L)] = zv
            return _

        lax.fori_loop(0, C, zrow_body, None)
        if with_counts:
            ov = jnp.ones((L,), f32)
            for j in range(C // L):
                ones_v[pl.ds(j * L, L)] = ov
            for j in range(RPS // L):
                zcnt_v[pl.ds(j * L, L)] = zv

        for b in range(RPS // C):
            pltpu.sync_copy(zrow_v, acc_sh.at[pl.ds(sid * RPS + b * C, C)])
        if with_counts:
            pltpu.sync_copy(zcnt_v, cnt_sh.at[pl.ds(sid * RPS, RPS)])
        plsc.subcore_barrier()

        pltpu.sync_copy(srcs.at[wid], src_v)
        pltpu.sync_copy(dsts.at[wid], dst_v)

        def chunk(k, _):
            pltpu.async_copy(table.at[src_v.at[k]], rows_v, semg).wait()
            pltpu.sync_copy(rows_v, acc_sh.at[dst_v.at[k]], add=True)
            if with_counts:
                pltpu.sync_copy(ones_v, cnt_sh.at[dst_v.at[k]], add=True)
            return _

        lax.fori_loop(0, K, chunk, None)
        plsc.subcore_barrier()

        rows = pl.ds(sid * RPS, RPS)
        pltpu.sync_copy(acc_sh.at[rows], out_sums.at[cid, rows])
        if with_counts:
            pltpu.sync_copy(cnt_sh.at[rows], out_cnts.at[cid, rows])

    return pl.kernel(body, out_type=tuple(out_type), mesh=mesh,
                     scratch_types=scratch,
                     compiler_params=pltpu.CompilerParams(
                         use_tc_tiling_on_sc=False))


_seg_sum_cnt = _make_seg_sum(with_counts=True)
_seg_sum = _make_seg_sum(with_counts=False)


_DN = (((1,), (1,)), ((), ()))


def _lin1_body(x_ref, wl_ref, wr_ref, xl_ref, xr_ref):
    x = x_ref[...]
    xl_ref[...] = lax.dot_general(x, wl_ref[...], _DN,
                                  preferred_element_type=f32)
    xr_ref[...] = lax.dot_general(x, wr_ref[...], _DN,
                                  preferred_element_type=f32)


def _mid_body(s_ref, c_ref, xr_ref, b1_ref, w2l_ref, w2r_ref,
              hl_ref, hr_ref):
    s = s_ref[0, pl.ds(0, N), :] + s_ref[1, pl.ds(0, N), :]
    c = jnp.maximum(c_ref[0, pl.ds(0, N)] + c_ref[1, pl.ds(0, N)], 1.0)
    h = jnp.maximum(s / c[:, None] + b1_ref[...] + xr_ref[...], 0.0)
    hl_ref[...] = lax.dot_general(h, w2l_ref[...], _DN,
                                  preferred_element_type=f32)
    hr_ref[...] = lax.dot_general(h, w2r_ref[...], _DN,
                                  preferred_element_type=f32)


def _head_body(s_ref, c_ref, hr_ref, b2_ref, wh_ref, bh_ref, out_ref):
    s = s_ref[0, pl.ds(0, N), :] + s_ref[1, pl.ds(0, N), :]
    c = jnp.maximum(c_ref[0, pl.ds(0, N)] + c_ref[1, pl.ds(0, N)], 1.0)
    h2 = jnp.maximum(s / c[:, None] + b2_ref[...] + hr_ref[...], 0.0)
    out_ref[...] = lax.dot_general(h2, wh_ref[...], _DN,
                                   preferred_element_type=f32) + bh_ref[0, 0]


def kernel(x, edge_index, W1l, b1l, W1r, W2l, b2l, W2r, Wh, bh):
    src = edge_index[0].astype(jnp.int32)
    dst = edge_index[1].astype(jnp.int32)
    pad = E_PAD - E
    src_p = jnp.concatenate([src, jnp.zeros((pad,), jnp.int32)])
    dst_p = jnp.concatenate([dst, jnp.full((pad,), N, jnp.int32)])
    srcs = src_p.reshape(NW, K, C)
    dsts = dst_p.reshape(NW, K, C)

    xl, xr = pl.pallas_call(
        _lin1_body,
        out_shape=(jax.ShapeDtypeStruct((N, HID), f32),
                   jax.ShapeDtypeStruct((N, HID), f32)),
    )(x, W1l, W1r)

    sums1, cnts = _seg_sum_cnt(xl, srcs, dsts)

    hl, hr = pl.pallas_call(
        _mid_body,
        out_shape=(jax.ShapeDtypeStruct((N, HID), f32),
                   jax.ShapeDtypeStruct((N, HID), f32)),
    )(sums1, cnts, xr, b1l.reshape(1, HID), W2l, W2r)

    (sums2,) = _seg_sum(hl, srcs, dsts)

    Wh_p = jnp.concatenate([Wh, jnp.zeros((7, HID), f32)])
    logits = pl.pallas_call(
        _head_body,
        out_shape=jax.ShapeDtypeStruct((N, 8), f32),
    )(sums2, cnts, hr, b2l.reshape(1, HID), Wh_p, bh.reshape(1, 1))

    return logits[:, 0]


# speedup vs baseline: 2.0681x; 1.5326x over previous
import functools

import jax
import jax.numpy as jnp
from jax import lax
from jax.experimental import pallas as pl
from jax.experimental.pallas import tpu as pltpu
from jax.experimental.pallas import tpu_sc as plsc

N = 10000
E = 320000
IN_DIM = 128
HID = 64

NC = 2
NS = 16
NW = NC * NS
L = 16

C = 128
K = 79
G = 2
SEG = 2 * G
EPW = K * C
E_PAD = NW * EPW
N_PAD = 10240
RPS = N_PAD // NS

f32 = jnp.float32


def _make_seg_sum(with_counts: bool):
    out_type = [jax.ShapeDtypeStruct((NC, N_PAD, HID), f32)]
    scratch = [
        pltpu.VMEM((K, C), jnp.int32),
        pltpu.VMEM((K, C), jnp.int32),
        pltpu.VMEM((C, HID), f32),
        pltpu.VMEM((C, HID), f32),
        pltpu.VMEM_SHARED((N_PAD, HID), f32),
        pltpu.SemaphoreType.DMA,
    ]
    if with_counts:
        out_type.append(jax.ShapeDtypeStruct((NC, N_PAD), f32))
        scratch += [
            pltpu.VMEM((C,), f32),
            pltpu.VMEM((RPS,), f32),
            pltpu.VMEM_SHARED((N_PAD,), f32),
        ]

    mesh = plsc.VectorSubcoreMesh(core_axis_name="c", subcore_axis_name="s")

    def body(table, srcs, dsts, *refs):
        if with_counts:
            (out_sums, out_cnts, src_v, dst_v, rows_v,
             zrow_v, acc_sh, semg, ones_v, zcnt_v, cnt_sh) = refs
        else:
            (out_sums, src_v, dst_v, rows_v,
             zrow_v, acc_sh, semg) = refs
        cid = lax.axis_index("c")
        sid = lax.axis_index("s")
        wid = sid * NC + cid

        zv = jnp.zeros((L,), f32)

        def zrow_body(i, _):
            for j in range(HID // L):
                zrow_v[i, pl.ds(j * L, L)] = zv
            return _

        lax.fori_loop(0, C, zrow_body, None)
        if with_counts:
            ov = jnp.ones((L,), f32)
            for j in range(C // L):
                ones_v[pl.ds(j * L, L)] = ov
            for j in range(RPS // L):
                zcnt_v[pl.ds(j * L, L)] = zv

        for b in range(RPS // C):
            pltpu.sync_copy(zrow_v, acc_sh.at[pl.ds(sid * RPS + b * C, C)])
        if with_counts:
            pltpu.sync_copy(zcnt_v, cnt_sh.at[pl.ds(sid * RPS, RPS)])
        plsc.subcore_barrier()

        pltpu.sync_copy(srcs.at[wid], src_v)
        pltpu.sync_copy(dsts.at[wid], dst_v)

        def chunk(k, _):
            pltpu.async_copy(table.at[src_v.at[k]], rows_v, semg).wait()
            pltpu.sync_copy(rows_v, acc_sh.at[dst_v.at[k]], add=True)
            if with_counts:
                pltpu.sync_copy(ones_v, cnt_sh.at[dst_v.at[k]], add=True)
            return _

        lax.fori_loop(0, K, chunk, None)
        plsc.subcore_barrier()

        rows = pl.ds(sid * RPS, RPS)
        pltpu.sync_copy(acc_sh.at[rows], out_sums.at[cid, rows])
        if with_counts:
            pltpu.sync_copy(cnt_sh.at[rows], out_cnts.at[cid, rows])

    return pl.kernel(body, out_type=tuple(out_type), mesh=mesh,
                     scratch_types=scratch,
                     compiler_params=pltpu.CompilerParams(
                         use_tc_tiling_on_sc=False))


_seg_sum_cnt = _make_seg_sum(with_counts=True)
_seg_sum = _make_seg_sum(with_counts=False)


_DN = (((1,), (1,)), ((), ()))


def _lin1_body(x_ref, wl_ref, wr_ref, xl_ref, xr_ref):
    x = x_ref[...]
    xl_ref[...] = lax.dot_general(x, wl_ref[...], _DN,
                                  preferred_element_type=f32)
    xr_ref[...] = lax.dot_general(x, wr_ref[...], _DN,
                                  preferred_element_type=f32)


def _mid_body(s_ref, c_ref, xr_ref, b1_ref, w2l_ref, w2r_ref,
              hl_ref, hr_ref):
    s = s_ref[0, pl.ds(0, N), :] + s_ref[1, pl.ds(0, N), :]
    c = jnp.maximum(c_ref[0, pl.ds(0, N)] + c_ref[1, pl.ds(0, N)], 1.0)
    h = jnp.maximum(s / c[:, None] + b1_ref[...] + xr_ref[...], 0.0)
    hl_ref[...] = lax.dot_general(h, w2l_ref[...], _DN,
                                  preferred_element_type=f32)
    hr_ref[...] = lax.dot_general(h, w2r_ref[...], _DN,
                                  preferred_element_type=f32)


def _head_body(s_ref, c_ref, hr_ref, b2_ref, wh_ref, bh_ref, out_ref):
    s = s_ref[0, pl.ds(0, N), :] + s_ref[1, pl.ds(0, N), :]
    c = jnp.maximum(c_ref[0, pl.ds(0, N)] + c_ref[1, pl.ds(0, N)], 1.0)
    h2 = jnp.maximum(s / c[:, None] + b2_ref[...] + hr_ref[...], 0.0)
    out_ref[...] = lax.dot_general(h2, wh_ref[...], _DN,
                                   preferred_element_type=f32) + bh_ref[0, 0]


def kernel(x, edge_index, W1l, b1l, W1r, W2l, b2l, W2r, Wh, bh):
    src = edge_index[0].astype(jnp.int32)
    dst = edge_index[1].astype(jnp.int32)
    pad = E_PAD - E
    pad_idx = jnp.arange(pad, dtype=jnp.int32)
    src_p = jnp.concatenate([src, pad_idx % N])
    dst_p = jnp.concatenate([dst, N + pad_idx % (N_PAD - N)])
    srcs = src_p.reshape(NW, K, C)
    dsts = dst_p.reshape(NW, K, C)

    xl, xr = pl.pallas_call(
        _lin1_body,
        out_shape=(jax.ShapeDtypeStruct((N, HID), f32),
                   jax.ShapeDtypeStruct((N, HID), f32)),
    )(x, W1l, W1r)

    sums1, cnts = _seg_sum_cnt(xl, srcs, dsts)

    hl, hr = pl.pallas_call(
        _mid_body,
        out_shape=(jax.ShapeDtypeStruct((N, HID), f32),
                   jax.ShapeDtypeStruct((N, HID), f32)),
    )(sums1, cnts, xr, b1l.reshape(1, HID), W2l, W2r)

    (sums2,) = _seg_sum(hl, srcs, dsts)

    Wh_p = jnp.concatenate([Wh, jnp.zeros((7, HID), f32)])
    logits = pl.pallas_call(
        _head_body,
        out_shape=jax.ShapeDtypeStruct((N, 8), f32),
    )(sums2, cnts, hr, b2l.reshape(1, HID), Wh_p, bh.reshape(1, 1))

    return logits[:, 0]


# speedup vs baseline: 2.9507x; 1.4267x over previous
import functools

import jax
import jax.numpy as jnp
from jax import lax
from jax.experimental import pallas as pl
from jax.experimental.pallas import tpu as pltpu
from jax.experimental.pallas import tpu_sc as plsc

N = 10000
E = 320000
IN_DIM = 128
HID = 64

NC = 2
NS = 16
NW = NC * NS
L = 16

C = 128
K = 80
G = 2
SEG = 2 * G
EPW = K * C
E_PAD = NW * EPW
N_PAD = 10240
RPS = N_PAD // NS

f32 = jnp.float32


def _make_seg_sum(with_counts: bool):
    out_type = [jax.ShapeDtypeStruct((NC, N_PAD, HID), f32)]
    scratch = [
        pltpu.VMEM((K, C), jnp.int32),
        pltpu.VMEM((K, C), jnp.int32),
    ] + [pltpu.VMEM((C, HID), f32) for _ in range(SEG)] + [
        pltpu.VMEM((C, HID), f32),
        pltpu.VMEM_SHARED((N_PAD, HID), f32),
        pltpu.SemaphoreType.DMA,
    ]
    if with_counts:
        out_type.append(jax.ShapeDtypeStruct((NC, N_PAD), f32))
        scratch += [
            pltpu.VMEM((C,), f32),
            pltpu.VMEM((RPS,), f32),
            pltpu.VMEM_SHARED((N_PAD,), f32),
        ]

    mesh = plsc.VectorSubcoreMesh(core_axis_name="c", subcore_axis_name="s")

    def body(table, srcs, dsts, *refs):
        if with_counts:
            (out_sums, out_cnts, src_v, dst_v, *rows,
             zrow_v, acc_sh, semg, ones_v, zcnt_v, cnt_sh) = refs
        else:
            (out_sums, src_v, dst_v, *rows,
             zrow_v, acc_sh, semg) = refs
        assert len(rows) == SEG
        cid = lax.axis_index("c")
        sid = lax.axis_index("s")
        wid = sid * NC + cid

        zv = jnp.zeros((L,), f32)

        def zrow_body(i, _):
            for j in range(HID // L):
                zrow_v[i, pl.ds(j * L, L)] = zv
            return _

        lax.fori_loop(0, C, zrow_body, None)
        if with_counts:
            ov = jnp.ones((L,), f32)
            for j in range(C // L):
                ones_v[pl.ds(j * L, L)] = ov
            for j in range(RPS // L):
                zcnt_v[pl.ds(j * L, L)] = zv

        for b in range(RPS // C):
            pltpu.sync_copy(zrow_v, acc_sh.at[pl.ds(sid * RPS + b * C, C)])
        if with_counts:
            pltpu.sync_copy(zcnt_v, cnt_sh.at[pl.ds(sid * RPS, RPS)])
        plsc.subcore_barrier()

        pltpu.sync_copy(srcs.at[wid], src_v)
        pltpu.sync_copy(dsts.at[wid], dst_v)

        def seg_body(i, _):
            base = i * SEG
            gd = [pltpu.async_copy(table.at[src_v.at[base + j]], rows[j],
                                   semg) for j in range(SEG)]
            for j in range(SEG):
                gd[j].wait()
                pltpu.sync_copy(rows[j], acc_sh.at[dst_v.at[base + j]],
                                add=True)
                if with_counts:
                    pltpu.sync_copy(ones_v, cnt_sh.at[dst_v.at[base + j]],
                                    add=True)
            return _

        lax.fori_loop(0, K // SEG, seg_body, None)
        plsc.subcore_barrier()

        rows = pl.ds(sid * RPS, RPS)
        pltpu.sync_copy(acc_sh.at[rows], out_sums.at[cid, rows])
        if with_counts:
            pltpu.sync_copy(cnt_sh.at[rows], out_cnts.at[cid, rows])

    return pl.kernel(body, out_type=tuple(out_type), mesh=mesh,
                     scratch_types=scratch,
                     compiler_params=pltpu.CompilerParams(
                         use_tc_tiling_on_sc=False))


_seg_sum_cnt = _make_seg_sum(with_counts=True)
_seg_sum = _make_seg_sum(with_counts=False)


_DN = (((1,), (1,)), ((), ()))


def _lin1_body(x_ref, wl_ref, wr_ref, xl_ref, xr_ref):
    x = x_ref[...]
    xl_ref[...] = lax.dot_general(x, wl_ref[...], _DN,
                                  preferred_element_type=f32)
    xr_ref[...] = lax.dot_general(x, wr_ref[...], _DN,
                                  preferred_element_type=f32)


def _mid_body(s_ref, c_ref, xr_ref, b1_ref, w2l_ref, w2r_ref,
              hl_ref, hr_ref):
    s = s_ref[0, pl.ds(0, N), :] + s_ref[1, pl.ds(0, N), :]
    c = jnp.maximum(c_ref[0, pl.ds(0, N)] + c_ref[1, pl.ds(0, N)], 1.0)
    h = jnp.maximum(s / c[:, None] + b1_ref[...] + xr_ref[...], 0.0)
    hl_ref[...] = lax.dot_general(h, w2l_ref[...], _DN,
                                  preferred_element_type=f32)
    hr_ref[...] = lax.dot_general(h, w2r_ref[...], _DN,
                                  preferred_element_type=f32)


def _head_body(s_ref, c_ref, hr_ref, b2_ref, wh_ref, bh_ref, out_ref):
    s = s_ref[0, pl.ds(0, N), :] + s_ref[1, pl.ds(0, N), :]
    c = jnp.maximum(c_ref[0, pl.ds(0, N)] + c_ref[1, pl.ds(0, N)], 1.0)
    h2 = jnp.maximum(s / c[:, None] + b2_ref[...] + hr_ref[...], 0.0)
    out_ref[...] = lax.dot_general(h2, wh_ref[...], _DN,
                                   preferred_element_type=f32) + bh_ref[0, 0]


def kernel(x, edge_index, W1l, b1l, W1r, W2l, b2l, W2r, Wh, bh):
    src = edge_index[0].astype(jnp.int32)
    dst = edge_index[1].astype(jnp.int32)
    pad = E_PAD - E
    pad_idx = jnp.arange(pad, dtype=jnp.int32)
    src_p = jnp.concatenate([src, pad_idx % N])
    dst_p = jnp.concatenate([dst, N + pad_idx % (N_PAD - N)])
    srcs = src_p.reshape(NW, K, C)
    dsts = dst_p.reshape(NW, K, C)

    xl, xr = pl.pallas_call(
        _lin1_body,
        out_shape=(jax.ShapeDtypeStruct((N, HID), f32),
                   jax.ShapeDtypeStruct((N, HID), f32)),
    )(x, W1l, W1r)

    sums1, cnts = _seg_sum_cnt(xl, srcs, dsts)

    hl, hr = pl.pallas_call(
        _mid_body,
        out_shape=(jax.ShapeDtypeStruct((N, HID), f32),
                   jax.ShapeDtypeStruct((N, HID), f32)),
    )(sums1, cnts, xr, b1l.reshape(1, HID), W2l, W2r)

    (sums2,) = _seg_sum(hl, srcs, dsts)

    Wh_p = jnp.concatenate([Wh, jnp.zeros((7, HID), f32)])
    logits = pl.pallas_call(
        _head_body,
        out_shape=jax.ShapeDtypeStruct((N, 8), f32),
    )(sums2, cnts, hr, b2l.reshape(1, HID), Wh_p, bh.reshape(1, 1))

    return logits[:, 0]


# speedup vs baseline: 3.0037x; 1.0179x over previous
import functools

import jax
import jax.numpy as jnp
from jax import lax
from jax.experimental import pallas as pl
from jax.experimental.pallas import tpu as pltpu
from jax.experimental.pallas import tpu_sc as plsc

N = 10000
E = 320000
IN_DIM = 128
HID = 64

NC = 2
NS = 16
NW = NC * NS
L = 16

C = 128
K = 80
SEG = 5
EPW = K * C
E_PAD = NW * EPW
N_PAD = 10240
RPS = N_PAD // NS

f32 = jnp.float32


def _make_seg_sum(with_counts: bool):
    out_type = [jax.ShapeDtypeStruct((NC, N_PAD, HID), f32)]
    scratch = [
        pltpu.VMEM((K, C), jnp.int32),
        pltpu.VMEM((K, C), jnp.int32),
    ] + [pltpu.VMEM((C, HID), f32) for _ in range(SEG)] + [
        pltpu.VMEM((C, HID), f32),
        pltpu.VMEM_SHARED((N_PAD, HID), f32),
        pltpu.SemaphoreType.DMA,
    ]
    if with_counts:
        out_type.append(jax.ShapeDtypeStruct((NC, N_PAD), f32))
        scratch += [
            pltpu.VMEM((C,), f32),
            pltpu.VMEM((RPS,), f32),
            pltpu.VMEM_SHARED((N_PAD,), f32),
        ]

    mesh = plsc.VectorSubcoreMesh(core_axis_name="c", subcore_axis_name="s")

    def body(table, srcs, dsts, *refs):
        if with_counts:
            (out_sums, out_cnts, src_v, dst_v, *rows,
             zrow_v, acc_sh, semg, ones_v, zcnt_v, cnt_sh) = refs
        else:
            (out_sums, src_v, dst_v, *rows,
             zrow_v, acc_sh, semg) = refs
        assert len(rows) == SEG
        cid = lax.axis_index("c")
        sid = lax.axis_index("s")
        wid = sid * NC + cid

        zv = jnp.zeros((L,), f32)

        def zrow_body(i, _):
            for j in range(HID // L):
                zrow_v[i, pl.ds(j * L, L)] = zv
            return _

        lax.fori_loop(0, C, zrow_body, None)
        if with_counts:
            ov = jnp.ones((L,), f32)
            for j in range(C // L):
                ones_v[pl.ds(j * L, L)] = ov
            for j in range(RPS // L):
                zcnt_v[pl.ds(j * L, L)] = zv

        for b in range(RPS // C):
            pltpu.sync_copy(zrow_v, acc_sh.at[pl.ds(sid * RPS + b * C, C)])
        if with_counts:
            pltpu.sync_copy(zcnt_v, cnt_sh.at[pl.ds(sid * RPS, RPS)])
        plsc.subcore_barrier()

        pltpu.sync_copy(srcs.at[wid], src_v)
        pltpu.sync_copy(dsts.at[wid], dst_v)

        def seg_body(i, _):
            base = i * SEG
            gd = [pltpu.async_copy(table.at[src_v.at[base + j]], rows[j],
                                   semg) for j in range(SEG)]
            for j in range(SEG):
                gd[j].wait()
                pltpu.sync_copy(rows[j], acc_sh.at[dst_v.at[base + j]],
                                add=True)
                if with_counts:
                    pltpu.sync_copy(ones_v, cnt_sh.at[dst_v.at[base + j]],
                                    add=True)
            return _

        lax.fori_loop(0, K // SEG, seg_body, None)
        plsc.subcore_barrier()

        rows = pl.ds(sid * RPS, RPS)
        pltpu.sync_copy(acc_sh.at[rows], out_sums.at[cid, rows])
        if with_counts:
            pltpu.sync_copy(cnt_sh.at[rows], out_cnts.at[cid, rows])

    return pl.kernel(body, out_type=tuple(out_type), mesh=mesh,
                     scratch_types=scratch,
                     compiler_params=pltpu.CompilerParams(
                         use_tc_tiling_on_sc=False))


_seg_sum_cnt = _make_seg_sum(with_counts=True)
_seg_sum = _make_seg_sum(with_counts=False)


_DN = (((1,), (1,)), ((), ()))


def _lin1_body(x_ref, wl_ref, wr_ref, xl_ref, xr_ref):
    x = x_ref[...]
    xl_ref[...] = lax.dot_general(x, wl_ref[...], _DN,
                                  preferred_element_type=f32)
    xr_ref[...] = lax.dot_general(x, wr_ref[...], _DN,
                                  preferred_element_type=f32)


def _mid_body(s_ref, c_ref, xr_ref, b1_ref, w2l_ref, w2r_ref,
              hl_ref, hr_ref):
    s = s_ref[0, pl.ds(0, N), :] + s_ref[1, pl.ds(0, N), :]
    c = jnp.maximum(c_ref[0, pl.ds(0, N)] + c_ref[1, pl.ds(0, N)], 1.0)
    h = jnp.maximum(s / c[:, None] + b1_ref[...] + xr_ref[...], 0.0)
    hl_ref[...] = lax.dot_general(h, w2l_ref[...], _DN,
                                  preferred_element_type=f32)
    hr_ref[...] = lax.dot_general(h, w2r_ref[...], _DN,
                                  preferred_element_type=f32)


def _head_body(s_ref, c_ref, hr_ref, b2_ref, wh_ref, bh_ref, out_ref):
    s = s_ref[0, pl.ds(0, N), :] + s_ref[1, pl.ds(0, N), :]
    c = jnp.maximum(c_ref[0, pl.ds(0, N)] + c_ref[1, pl.ds(0, N)], 1.0)
    h2 = jnp.maximum(s / c[:, None] + b2_ref[...] + hr_ref[...], 0.0)
    out_ref[...] = lax.dot_general(h2, wh_ref[...], _DN,
                                   preferred_element_type=f32) + bh_ref[0, 0]


def kernel(x, edge_index, W1l, b1l, W1r, W2l, b2l, W2r, Wh, bh):
    src = edge_index[0].astype(jnp.int32)
    dst = edge_index[1].astype(jnp.int32)
    pad = E_PAD - E
    pad_idx = jnp.arange(pad, dtype=jnp.int32)
    src_p = jnp.concatenate([src, pad_idx % N])
    dst_p = jnp.concatenate([dst, N + pad_idx % (N_PAD - N)])
    srcs = src_p.reshape(NW, K, C)
    dsts = dst_p.reshape(NW, K, C)

    xl, xr = pl.pallas_call(
        _lin1_body,
        out_shape=(jax.ShapeDtypeStruct((N, HID), f32),
                   jax.ShapeDtypeStruct((N, HID), f32)),
    )(x, W1l, W1r)

    sums1, cnts = _seg_sum_cnt(xl, srcs, dsts)

    hl, hr = pl.pallas_call(
        _mid_body,
        out_shape=(jax.ShapeDtypeStruct((N, HID), f32),
                   jax.ShapeDtypeStruct((N, HID), f32)),
    )(sums1, cnts, xr, b1l.reshape(1, HID), W2l, W2r)

    (sums2,) = _seg_sum(hl, srcs, dsts)

    Wh_p = jnp.concatenate([Wh, jnp.zeros((7, HID), f32)])
    logits = pl.pallas_call(
        _head_body,
        out_shape=jax.ShapeDtypeStruct((N, 8), f32),
    )(sums2, cnts, hr, b2l.reshape(1, HID), Wh_p, bh.reshape(1, 1))

    return logits[:, 0]


# speedup vs baseline: 3.1751x; 1.0571x over previous
import jax
import jax.numpy as jnp
from jax import lax
from jax.experimental import pallas as pl
from jax.experimental.pallas import tpu as pltpu
from jax.experimental.pallas import tpu_sc as plsc

N = 10000
E = 320000
IN_DIM = 128
HID = 64

NC = 2
NS = 16
NW = NC * NS
L = 16

C = 128
TCH = E // C
KBASE = TCH // NW
EXTRA = TCH - KBASE * NW
KMAX = KBASE + 1
SEG = 6
SEGS = KBASE // SEG
N_PAD = 10240
RPS = N_PAD // NS

f32 = jnp.float32


def _make_seg_sum(with_counts: bool):
    out_type = [jax.ShapeDtypeStruct((NC, N_PAD, HID), f32)]
    scratch = [
        pltpu.VMEM((KMAX, C), jnp.int32),
        pltpu.VMEM((KMAX, C), jnp.int32),
    ] + [pltpu.VMEM((C, HID), f32) for _ in range(SEG)] + [
        pltpu.VMEM((C, HID), f32),
        pltpu.VMEM_SHARED((N_PAD, HID), f32),
        pltpu.SemaphoreType.DMA,
    ]
    if with_counts:
        out_type.append(jax.ShapeDtypeStruct((NC, N_PAD), f32))
        scratch += [
            pltpu.VMEM((C,), f32),
            pltpu.VMEM((RPS,), f32),
            pltpu.VMEM_SHARED((N_PAD,), f32),
        ]

    mesh = plsc.VectorSubcoreMesh(core_axis_name="c", subcore_axis_name="s")

    def body(table, edges, *refs):
        if with_counts:
            (out_sums, out_cnts, src_v, dst_v, *rows,
             zrow_v, acc_sh, semg, ones_v, zcnt_v, cnt_sh) = refs
        else:
            (out_sums, src_v, dst_v, *rows,
             zrow_v, acc_sh, semg) = refs
        assert len(rows) == SEG
        cid = lax.axis_index("c")
        sid = lax.axis_index("s")
        wid = sid * NC + cid
        start = KBASE * wid + jnp.minimum(wid, EXTRA)
        has_extra = wid < EXTRA

        zv = jnp.zeros((L,), f32)

        def zrow_body(i, _):
            for j in range(HID // L):
                zrow_v[i, pl.ds(j * L, L)] = zv
            return _

        lax.fori_loop(0, C, zrow_body, None)
        if with_counts:
            ov = jnp.ones((L,), f32)
            for j in range(C // L):
                ones_v[pl.ds(j * L, L)] = ov
            for j in range(RPS // L):
                zcnt_v[pl.ds(j * L, L)] = zv

        for b in range(RPS // C):
            pltpu.sync_copy(zrow_v, acc_sh.at[pl.ds(sid * RPS + b * C, C)])
        if with_counts:
            pltpu.sync_copy(zcnt_v, cnt_sh.at[pl.ds(sid * RPS, RPS)])
        plsc.subcore_barrier()

        pltpu.sync_copy(edges.at[0, pl.ds(start, KBASE)],
                        src_v.at[pl.ds(0, KBASE)])
        pltpu.sync_copy(edges.at[1, pl.ds(start, KBASE)],
                        dst_v.at[pl.ds(0, KBASE)])

        @pl.when(has_extra)
        def _():
            pltpu.sync_copy(edges.at[0, pl.ds(start + KBASE, 1)],
                            src_v.at[pl.ds(KBASE, 1)])
            pltpu.sync_copy(edges.at[1, pl.ds(start + KBASE, 1)],
                            dst_v.at[pl.ds(KBASE, 1)])

        def do_chunk(k, buf):
            pltpu.sync_copy(buf, acc_sh.at[dst_v.at[k]], add=True)
            if with_counts:
                pltpu.sync_copy(ones_v, cnt_sh.at[dst_v.at[k]], add=True)

        def seg_body(i, _):
            base = i * SEG
            gd = [pltpu.async_copy(table.at[src_v.at[base + j]], rows[j],
                                   semg) for j in range(SEG)]
            for j in range(SEG):
                gd[j].wait()
                do_chunk(base + j, rows[j])
            return _

        lax.fori_loop(0, SEGS, seg_body, None)

        @pl.when(has_extra)
        def _():
            pltpu.async_copy(table.at[src_v.at[KBASE]], rows[0],
                             semg).wait()
            do_chunk(KBASE, rows[0])

        plsc.subcore_barrier()

        rslice = pl.ds(sid * RPS, RPS)
        pltpu.sync_copy(acc_sh.at[rslice], out_sums.at[cid, rslice])
        if with_counts:
            pltpu.sync_copy(cnt_sh.at[rslice], out_cnts.at[cid, rslice])

    return pl.kernel(body, out_type=tuple(out_type), mesh=mesh,
                     scratch_types=scratch,
                     compiler_params=pltpu.CompilerParams(
                         use_tc_tiling_on_sc=False))


_seg_sum_cnt = _make_seg_sum(with_counts=True)
_seg_sum = _make_seg_sum(with_counts=False)


_DN = (((1,), (1,)), ((), ()))


def _lin1_body(x_ref, wl_ref, wr_ref, xl_ref, xr_ref):
    x = x_ref[...]
    xl_ref[...] = lax.dot_general(x, wl_ref[...], _DN,
                                  preferred_element_type=f32)
    xr_ref[...] = lax.dot_general(x, wr_ref[...], _DN,
                                  preferred_element_type=f32)


def _mid_body(s_ref, c_ref, xr_ref, b1_ref, h_ref):
    s = s_ref[0, pl.ds(0, N), :] + s_ref[1, pl.ds(0, N), :]
    c = jnp.maximum(c_ref[0, pl.ds(0, N)] + c_ref[1, pl.ds(0, N)], 1.0)
    h_ref[...] = jnp.maximum(s / c[:, None] + b1_ref[...] + xr_ref[...], 0.0)


def _head_body(s_ref, c_ref, h_ref, b2_ref, w2l_ref, w2r_ref, wh_ref,
               bh_ref, out_ref):
    s = s_ref[0, pl.ds(0, N), :] + s_ref[1, pl.ds(0, N), :]
    c = jnp.maximum(c_ref[0, pl.ds(0, N)] + c_ref[1, pl.ds(0, N)], 1.0)
    mean2 = s / c[:, None]
    h2 = jnp.maximum(
        lax.dot_general(mean2, w2l_ref[...], _DN, preferred_element_type=f32)
        + b2_ref[...]
        + lax.dot_general(h_ref[...], w2r_ref[...], _DN,
                          preferred_element_type=f32),
        0.0)
    out_ref[...] = lax.dot_general(h2, wh_ref[...], _DN,
                                   preferred_element_type=f32) + bh_ref[0, 0]


def kernel(x, edge_index, W1l, b1l, W1r, W2l, b2l, W2r, Wh, bh):
    edges = edge_index.astype(jnp.int32).reshape(2, TCH, C)

    xl, xr = pl.pallas_call(
        _lin1_body,
        out_shape=(jax.ShapeDtypeStruct((N, HID), f32),
                   jax.ShapeDtypeStruct((N, HID), f32)),
    )(x, W1l, W1r)

    sums1, cnts = _seg_sum_cnt(xl, edges)

    h = pl.pallas_call(
        _mid_body,
        out_shape=jax.ShapeDtypeStruct((N, HID), f32),
    )(sums1, cnts, xr, b1l.reshape(1, HID))

    (sums2,) = _seg_sum(h, edges)

    Wh_p = jnp.concatenate([Wh, jnp.zeros((7, HID), f32)])
    logits = pl.pallas_call(
        _head_body,
        out_shape=jax.ShapeDtypeStruct((N, 8), f32),
    )(sums2, cnts, h, b2l.reshape(1, HID), W2l, W2r, Wh_p, bh.reshape(1, 1))

    return logits[:, 0]


# speedup vs baseline: 3.2630x; 1.0277x over previous
import jax
import jax.numpy as jnp
from jax import lax
from jax.experimental import pallas as pl
from jax.experimental.pallas import tpu as pltpu
from jax.experimental.pallas import tpu_sc as plsc

N = 10000
E = 320000
IN_DIM = 128
HID = 64

NC = 2
NS = 16
NW = NC * NS
L = 16

C = 128
TCH = E // C
KBASE = TCH // NW
EXTRA = TCH - KBASE * NW
KMAX = KBASE + 1
SEG = 6
SEGS = KBASE // SEG
N_PAD = 10240
RPS = N_PAD // NS

f32 = jnp.float32


def _make_seg_sum(with_counts: bool):
    out_type = [jax.ShapeDtypeStruct((NC, N_PAD, HID), f32)]
    scratch = [
        pltpu.VMEM((KMAX, C), jnp.int32),
        pltpu.VMEM((KMAX, C), jnp.int32),
    ] + [pltpu.VMEM((C, HID), f32) for _ in range(SEG)] + [
        pltpu.VMEM((C, HID), f32),
        pltpu.VMEM_SHARED((N_PAD, HID), f32),
        pltpu.SemaphoreType.DMA,
        pltpu.SemaphoreType.DMA,
    ]
    if with_counts:
        out_type.append(jax.ShapeDtypeStruct((NC, N_PAD), f32))
        scratch += [
            pltpu.VMEM((C,), f32),
            pltpu.VMEM((RPS,), f32),
            pltpu.VMEM_SHARED((N_PAD,), f32),
        ]

    mesh = plsc.VectorSubcoreMesh(core_axis_name="c", subcore_axis_name="s")

    def body(table, edges, *refs):
        if with_counts:
            (out_sums, out_cnts, src_v, dst_v, *rows,
             zrow_v, acc_sh, semg, semv, ones_v, zcnt_v, cnt_sh) = refs
        else:
            (out_sums, src_v, dst_v, *rows,
             zrow_v, acc_sh, semg, semv) = refs
        assert len(rows) == SEG
        cid = lax.axis_index("c")
        sid = lax.axis_index("s")
        wid = sid * NC + cid
        start = KBASE * wid + jnp.minimum(wid, EXTRA)
        has_extra = wid < EXTRA

        zv = jnp.zeros((L,), f32)

        def zrow_body(i, _):
            for j in range(HID // L):
                zrow_v[i, pl.ds(j * L, L)] = zv
            return _

        lax.fori_loop(0, C, zrow_body, None)
        if with_counts:
            ov = jnp.ones((L,), f32)
            for j in range(C // L):
                ones_v[pl.ds(j * L, L)] = ov
            for j in range(RPS // L):
                zcnt_v[pl.ds(j * L, L)] = zv

        for b in range(RPS // C):
            pltpu.sync_copy(zrow_v, acc_sh.at[pl.ds(sid * RPS + b * C, C)])
        if with_counts:
            pltpu.sync_copy(zcnt_v, cnt_sh.at[pl.ds(sid * RPS, RPS)])
        plsc.subcore_barrier()

        pltpu.sync_copy(edges.at[0, pl.ds(start, KBASE)],
                        src_v.at[pl.ds(0, KBASE)])
        pltpu.sync_copy(edges.at[1, pl.ds(start, KBASE)],
                        dst_v.at[pl.ds(0, KBASE)])

        @pl.when(has_extra)
        def _():
            pltpu.sync_copy(edges.at[0, pl.ds(start + KBASE, 1)],
                            src_v.at[pl.ds(KBASE, 1)])
            pltpu.sync_copy(edges.at[1, pl.ds(start + KBASE, 1)],
                            dst_v.at[pl.ds(KBASE, 1)])

        def do_chunk(k, buf):
            descs = [pltpu.async_copy(buf, acc_sh.at[dst_v.at[k]], semv,
                                      add=True)]
            if with_counts:
                descs.append(pltpu.async_copy(ones_v, cnt_sh.at[dst_v.at[k]],
                                              semv, add=True))
            return descs

        def seg_body(i, _):
            base = i * SEG
            gd = [pltpu.async_copy(table.at[src_v.at[base + j]], rows[j],
                                   semg) for j in range(SEG)]
            sd = []
            for j in range(SEG):
                gd[j].wait()
                sd += do_chunk(base + j, rows[j])
            for d in sd:
                d.wait()
            return _

        lax.fori_loop(0, SEGS, seg_body, None)

        @pl.when(has_extra)
        def _():
            pltpu.async_copy(table.at[src_v.at[KBASE]], rows[0],
                             semg).wait()
            for d in do_chunk(KBASE, rows[0]):
                d.wait()

        plsc.subcore_barrier()

        rslice = pl.ds(sid * RPS, RPS)
        pltpu.sync_copy(acc_sh.at[rslice], out_sums.at[cid, rslice])
        if with_counts:
            pltpu.sync_copy(cnt_sh.at[rslice], out_cnts.at[cid, rslice])

    return pl.kernel(body, out_type=tuple(out_type), mesh=mesh,
                     scratch_types=scratch,
                     compiler_params=pltpu.CompilerParams(
                         use_tc_tiling_on_sc=False))


_seg_sum_cnt = _make_seg_sum(with_counts=True)
_seg_sum = _make_seg_sum(with_counts=False)


_DN = (((1,), (1,)), ((), ()))


def _lin1_body(x_ref, wl_ref, wr_ref, xl_ref, xr_ref):
    x = x_ref[...]
    xl_ref[...] = lax.dot_general(x, wl_ref[...], _DN,
                                  preferred_element_type=f32)
    xr_ref[...] = lax.dot_general(x, wr_ref[...], _DN,
                                  preferred_element_type=f32)


def _mid_body(s_ref, c_ref, xr_ref, b1_ref, h_ref):
    s = s_ref[0, pl.ds(0, N), :] + s_ref[1, pl.ds(0, N), :]
    c = jnp.maximum(c_ref[0, pl.ds(0, N)] + c_ref[1, pl.ds(0, N)], 1.0)
    h_ref[...] = jnp.maximum(s / c[:, None] + b1_ref[...] + xr_ref[...], 0.0)


def _head_body(s_ref, c_ref, h_ref, b2_ref, w2l_ref, w2r_ref, wh_ref,
               bh_ref, out_ref):
    s = s_ref[0, pl.ds(0, N), :] + s_ref[1, pl.ds(0, N), :]
    c = jnp.maximum(c_ref[0, pl.ds(0, N)] + c_ref[1, pl.ds(0, N)], 1.0)
    mean2 = s / c[:, None]
    h2 = jnp.maximum(
        lax.dot_general(mean2, w2l_ref[...], _DN, preferred_element_type=f32)
        + b2_ref[...]
        + lax.dot_general(h_ref[...], w2r_ref[...], _DN,
                          preferred_element_type=f32),
        0.0)
    out_ref[...] = lax.dot_general(h2, wh_ref[...], _DN,
                                   preferred_element_type=f32) + bh_ref[0, 0]


def kernel(x, edge_index, W1l, b1l, W1r, W2l, b2l, W2r, Wh, bh):
    edges = edge_index.astype(jnp.int32).reshape(2, TCH, C)

    xl, xr = pl.pallas_call(
        _lin1_body,
        out_shape=(jax.ShapeDtypeStruct((N, HID), f32),
                   jax.ShapeDtypeStruct((N, HID), f32)),
    )(x, W1l, W1r)

    sums1, cnts = _seg_sum_cnt(xl, edges)

    h = pl.pallas_call(
        _mid_body,
        out_shape=jax.ShapeDtypeStruct((N, HID), f32),
    )(sums1, cnts, xr, b1l.reshape(1, HID))

    (sums2,) = _seg_sum(h, edges)

    Wh_p = jnp.concatenate([Wh, jnp.zeros((7, HID), f32)])
    logits = pl.pallas_call(
        _head_body,
        out_shape=jax.ShapeDtypeStruct((N, 8), f32),
    )(sums2, cnts, h, b2l.reshape(1, HID), W2l, W2r, Wh_p, bh.reshape(1, 1))

    return logits[:, 0]


# speedup vs baseline: 3.2735x; 1.0032x over previous
import jax
import jax.numpy as jnp
from jax import lax
from jax.experimental import pallas as pl
from jax.experimental.pallas import tpu as pltpu
from jax.experimental.pallas import tpu_sc as plsc

N = 10000
E = 320000
IN_DIM = 128
HID = 64

NC = 2
NS = 16
NW = NC * NS
L = 16

C = 128
TCH = E // C
KBASE = TCH // NW
EXTRA = TCH - KBASE * NW
KMAX = KBASE + 1
SEG = 6
SEGS = KBASE // SEG
N_PAD = 10240
RPS = N_PAD // NS

f32 = jnp.float32


def _make_seg_sum(with_counts: bool):
    out_type = [jax.ShapeDtypeStruct((NC, N_PAD, HID), f32)]
    scratch = [
        pltpu.VMEM((KMAX, C), jnp.int32),
        pltpu.VMEM((KMAX, C), jnp.int32),
    ] + [pltpu.VMEM((C, HID), f32) for _ in range(SEG)] + [
        pltpu.VMEM((C, HID), f32),
        pltpu.VMEM_SHARED((N_PAD, HID), f32),
        pltpu.SemaphoreType.DMA,
        pltpu.SemaphoreType.DMA,
    ]
    if with_counts:
        out_type.append(jax.ShapeDtypeStruct((NC, N_PAD), f32))
        scratch += [
            pltpu.VMEM((C,), f32),
            pltpu.VMEM((RPS,), f32),
            pltpu.VMEM_SHARED((N_PAD,), f32),
        ]

    mesh = plsc.VectorSubcoreMesh(core_axis_name="c", subcore_axis_name="s")

    def body(table, edges, *refs):
        if with_counts:
            (out_sums, out_cnts, src_v, dst_v, *rows,
             zrow_v, acc_sh, semg, semv, ones_v, zcnt_v, cnt_sh) = refs
        else:
            (out_sums, src_v, dst_v, *rows,
             zrow_v, acc_sh, semg, semv) = refs
        assert len(rows) == SEG
        cid = lax.axis_index("c")
        sid = lax.axis_index("s")
        wid = sid * NC + cid
        start = KBASE * wid + jnp.minimum(wid, EXTRA)
        has_extra = wid < EXTRA

        zv = jnp.zeros((L,), f32)

        def zrow_body(i, _):
            for j in range(HID // L):
                zrow_v[i, pl.ds(j * L, L)] = zv
            return _

        lax.fori_loop(0, C, zrow_body, None)
        if with_counts:
            ov = jnp.ones((L,), f32)
            for j in range(C // L):
                ones_v[pl.ds(j * L, L)] = ov
            for j in range(RPS // L):
                zcnt_v[pl.ds(j * L, L)] = zv

        for b in range(RPS // C):
            pltpu.sync_copy(zrow_v, acc_sh.at[pl.ds(sid * RPS + b * C, C)])
        if with_counts:
            pltpu.sync_copy(zcnt_v, cnt_sh.at[pl.ds(sid * RPS, RPS)])
        plsc.subcore_barrier()

        pltpu.sync_copy(edges.at[0, pl.ds(start, KBASE)],
                        src_v.at[pl.ds(0, KBASE)])
        pltpu.sync_copy(edges.at[1, pl.ds(start, KBASE)],
                        dst_v.at[pl.ds(0, KBASE)])

        @pl.when(has_extra)
        def _():
            pltpu.sync_copy(edges.at[0, pl.ds(start + KBASE, 1)],
                            src_v.at[pl.ds(KBASE, 1)])
            pltpu.sync_copy(edges.at[1, pl.ds(start + KBASE, 1)],
                            dst_v.at[pl.ds(KBASE, 1)])

        def do_chunk(k, buf):
            descs = [pltpu.async_copy(buf, acc_sh.at[dst_v.at[k]], semv,
                                      add=True)]
            if with_counts:
                descs.append(pltpu.async_copy(ones_v, cnt_sh.at[dst_v.at[k]],
                                              semv, add=True))
            return descs

        def seg_body(i, _):
            base = i * SEG
            gd = [pltpu.async_copy(table.at[src_v.at[base + j]], rows[j],
                                   semg) for j in range(SEG)]
            sd = []
            for j in range(SEG):
                gd[j].wait()
                sd += do_chunk(base + j, rows[j])
            for d in sd:
                d.wait()
            return _

        lax.fori_loop(0, SEGS, seg_body, None)

        @pl.when(has_extra)
        def _():
            pltpu.async_copy(table.at[src_v.at[KBASE]], rows[0],
                             semg).wait()
            for d in do_chunk(KBASE, rows[0]):
                d.wait()

        plsc.subcore_barrier()

        rslice = pl.ds(sid * RPS, RPS)
        pltpu.sync_copy(acc_sh.at[rslice], out_sums.at[cid, rslice])
        if with_counts:
            pltpu.sync_copy(cnt_sh.at[rslice], out_cnts.at[cid, rslice])

    return pl.kernel(body, out_type=tuple(out_type), mesh=mesh,
                     scratch_types=scratch,
                     compiler_params=pltpu.CompilerParams(
                         use_tc_tiling_on_sc=False))


_seg_sum_cnt = _make_seg_sum(with_counts=True)
_seg_sum = _make_seg_sum(with_counts=False)


_DN = (((1,), (1,)), ((), ()))


def _matmul_body(x_ref, w_ref, out_ref):
    out_ref[...] = lax.dot_general(x_ref[...], w_ref[...], _DN,
                                   preferred_element_type=f32)


def _matmul(x, w):
    return pl.pallas_call(
        _matmul_body,
        out_shape=jax.ShapeDtypeStruct((x.shape[0], w.shape[0]), f32),
    )(x, w)


def _mid_body(s_ref, c_ref, xr_ref, b1_ref, h_ref):
    s = s_ref[0, pl.ds(0, N), :] + s_ref[1, pl.ds(0, N), :]
    c = jnp.maximum(c_ref[0, pl.ds(0, N)] + c_ref[1, pl.ds(0, N)], 1.0)
    h_ref[...] = jnp.maximum(s / c[:, None] + b1_ref[...] + xr_ref[...], 0.0)


def _head_body(s_ref, c_ref, hr_ref, b2_ref, w2l_ref, wh_ref,
               bh_ref, out_ref):
    s = s_ref[0, pl.ds(0, N), :] + s_ref[1, pl.ds(0, N), :]
    c = jnp.maximum(c_ref[0, pl.ds(0, N)] + c_ref[1, pl.ds(0, N)], 1.0)
    mean2 = s / c[:, None]
    h2 = jnp.maximum(
        lax.dot_general(mean2, w2l_ref[...], _DN, preferred_element_type=f32)
        + b2_ref[...] + hr_ref[...],
        0.0)
    out_ref[...] = lax.dot_general(h2, wh_ref[...], _DN,
                                   preferred_element_type=f32) + bh_ref[0, 0]


def kernel(x, edge_index, W1l, b1l, W1r, W2l, b2l, W2r, Wh, bh):
    edges = edge_index.astype(jnp.int32).reshape(2, TCH, C)

    xl = _matmul(x, W1l)
    sums1, cnts = _seg_sum_cnt(xl, edges)
    xr = _matmul(x, W1r)

    h = pl.pallas_call(
        _mid_body,
        out_shape=jax.ShapeDtypeStruct((N, HID), f32),
    )(sums1, cnts, xr, b1l.reshape(1, HID))

    (sums2,) = _seg_sum(h, edges)
    hr = _matmul(h, W2r)

    Wh_p = jnp.concatenate([Wh, jnp.zeros((7, HID), f32)])
    logits = pl.pallas_call(
        _head_body,
        out_shape=jax.ShapeDtypeStruct((N, 8), f32),
    )(sums2, cnts, hr, b2l.reshape(1, HID), W2l, Wh_p, bh.reshape(1, 1))

    return logits[:, 0]
